# baseline (speedup 1.0000x reference)
import jax
import jax.numpy as jnp
from jax import lax
from jax.experimental import pallas as pl
from jax.experimental.pallas import tpu as pltpu

_ALPHA = 0.3
_BETA = 0.5
_TB = 128
_PCH = 512
_ADDR_A = 128
_ADDR_B = 160


def _sigmoid(x):
    return 0.5 * jnp.tanh(0.5 * x) + 0.5


def _acc2(addr, lhs, w_ref, staged):
    pltpu.matmul_push_rhs(w_ref[0], staging_register=staged, mxu_index=0)
    pltpu.matmul_push_rhs(w_ref[1], staging_register=staged, mxu_index=1)
    pltpu.matmul_acc_lhs(addr, lhs, mxu_index=0, load_staged_rhs=staged)
    pltpu.matmul_acc_lhs(addr, lhs, mxu_index=1, load_staged_rhs=staged)


def _pop2(addr, m):
    g0 = pltpu.matmul_pop(addr, (m, 256), jnp.float32, mxu_index=0)
    g1 = pltpu.matmul_pop(addr, (m, 256), jnp.float32, mxu_index=1)
    return g0, g1


def _fused_kernel(x_ref, wih_ref, whh_ref, bproj_ref, bhhn_ref, wfc_ref,
                  bfc_ref, out_ref, ema1_ref, ema2_ref, h_ref, xs2_ref,
                  xp_ref):
    tb = pl.program_id(1)
    nb = pl.num_programs(1) - 1
    bm = x_ref.shape[1]
    zpad = jnp.zeros((bm, 128), jnp.float32)

    @pl.when(tb == 0)
    def _init():
        h_ref[...] = jnp.zeros_like(h_ref)
        xs2_ref[...] = jnp.zeros_like(xs2_ref)
        x0 = x_ref[0]
        ema1_ref[...] = x0
        ema2_ref[...] = x0

    sel_w = lax.rem(tb, 2)
    sel_r = lax.rem(tb + 1, 2)

    e1_in = ema1_ref[...]
    e2_in = ema2_ref[...]

    _UN = 8

    def body(i, carry):
        prev1, prev2, h = carry
        t0 = i * _UN
        _acc2(_ADDR_A, jnp.concatenate([h, zpad], axis=1), whh_ref, 0)
        for j in range(_UN):
            t = t0 + j
            gh0, gh1 = _pop2(_ADDR_A, bm)
            gi = xp_ref[sel_r, pl.ds(t * bm, bm), :]
            r = _sigmoid(gi[:, 0:128] + gh0[:, 0:128])
            z = _sigmoid(gi[:, 128:256] + gh0[:, 128:256])
            n = jnp.tanh(gi[:, 256:384] + r * (gh1[:, 0:128] + bhhn_ref[...]))
            h = n + z * (h - n)
            if j < _UN - 1:
                _acc2(_ADDR_A, jnp.concatenate([h, zpad], axis=1), whh_ref, 0)
            xt = x_ref[t]
            cur1 = (1.0 - _ALPHA) * xt + _ALPHA * prev1
            cur2 = (1.0 - _BETA) * cur1 + _BETA * prev2
            xs2_ref[t, :, 0:64] = cur2
            prev1, prev2 = cur1, cur2
        return (prev1, prev2, h)

    p1, p2, h = lax.fori_loop(
        0, _TB // _UN, body, (e1_in, e2_in, h_ref[...]))

    @pl.when(tb < nb)
    def _commit_ema():
        ema1_ref[...] = p1
        ema2_ref[...] = p2

    h_ref[...] = h

    @pl.when(tb == 0)
    def _reset_h():
        h_ref[...] = jnp.zeros_like(h_ref)

    @pl.when(tb < nb)
    def _project():
        tpc = _PCH // bm
        for k in range(_TB // tpc):
            lhs = xs2_ref[k * tpc:(k + 1) * tpc].reshape(_PCH, 256)
            _acc2(0, lhs, wih_ref, 1)
            u0, u1 = _pop2(0, _PCH)
            rows = pl.ds(k * _PCH, _PCH)
            xp_ref[sel_w, rows, 0:256] = u0 + bproj_ref[:, 0:256]
            xp_ref[sel_w, rows, 256:384] = u1[:, 0:128] + bproj_ref[:, 256:384]

    @pl.when(tb == nb)
    def _final():
        pltpu.matmul_push_rhs(wfc_ref[...], staging_register=1, mxu_index=0)
        lhs = jnp.concatenate([h, zpad], axis=1)
        pltpu.matmul_acc_lhs(0, lhs, mxu_index=0, load_staged_rhs=1)
        out = pltpu.matmul_pop(0, (bm, 256), jnp.float32, mxu_index=0)
        out = out[:, 0:2] + bfc_ref[...]
        st11 = e1_in[:, 1:3]
        st12 = e2_in[:, 1:3]
        out = (out - _BETA * st12) / (1.0 - _BETA)
        out = (out - _ALPHA * st11) / (1.0 - _ALPHA)
        out_ref[...] = out


def kernel(x, W_ih, W_hh, b_ih, b_hh, W_fc, b_fc):
    B, T, I = x.shape
    H = W_hh.shape[1]
    C = W_fc.shape[0]
    nb = T // _TB
    xt = jnp.swapaxes(x, 0, 1)

    def stage(Wt, k):
        Wp = jnp.pad(Wt, ((0, 256 - k), (0, 2 * 256 - 3 * H)))
        return jnp.stack([Wp[:, 0:256], Wp[:, 256:512]])

    whh_s = stage(W_hh.T, H)
    wih_s = stage(W_ih.T, I)
    wfc_s = jnp.pad(W_fc.T, ((0, 256 - H), (0, 256 - C)))

    b_proj = (b_ih + jnp.concatenate([b_hh[:2 * H], jnp.zeros((H,), b_hh.dtype)]))
    b_hh_n = b_hh[2 * H:]

    out = pl.pallas_call(
        _fused_kernel,
        grid=(2, nb + 1),
        in_specs=[
            pl.BlockSpec((_TB, B // 2, I),
                         lambda c, t: (jnp.minimum(t, nb - 1), c, 0)),
            pl.BlockSpec((2, 256, 256), lambda c, t: (0, 0, 0)),
            pl.BlockSpec((2, 256, 256), lambda c, t: (0, 0, 0)),
            pl.BlockSpec((1, 3 * H), lambda c, t: (0, 0)),
            pl.BlockSpec((1, H), lambda c, t: (0, 0)),
            pl.BlockSpec((256, 256), lambda c, t: (0, 0)),
            pl.BlockSpec((1, C), lambda c, t: (0, 0)),
        ],
        out_specs=pl.BlockSpec((B // 2, C), lambda c, t: (c, 0)),
        out_shape=jax.ShapeDtypeStruct((B, C), jnp.float32),
        scratch_shapes=[
            pltpu.VMEM((B // 2, I), jnp.float32),
            pltpu.VMEM((B // 2, I), jnp.float32),
            pltpu.VMEM((B // 2, H), jnp.float32),
            pltpu.VMEM((_TB, B // 2, 256), jnp.float32),
            pltpu.VMEM((2, _TB * (B // 2), 3 * H), jnp.float32),
        ],
        compiler_params=pltpu.CompilerParams(
            dimension_semantics=("parallel", "arbitrary"),
            vmem_limit_bytes=100 * 1024 * 1024,
        ),
    )(xt, wih_s, whh_s, b_proj.reshape(1, -1), b_hh_n.reshape(1, -1),
      wfc_s, b_fc.reshape(1, -1))

    return out[:, None, :]


# speedup vs baseline: 1.0730x; 1.0730x over previous
import jax
import jax.numpy as jnp
from jax import lax
from jax.experimental import pallas as pl
from jax.experimental.pallas import tpu as pltpu

_ALPHA = 0.3
_BETA = 0.5
_TB = 128


def _sigmoid(x):
    return 0.5 * jnp.tanh(0.5 * x) + 0.5


def _fused_kernel(x_ref, wih_ref, whh_ref, bproj_ref, bhhn_ref, wfc_ref,
                  bfc_ref, out_ref, ema1_ref, ema2_ref, h_ref, xs2_ref,
                  xp_ref):
    tb = pl.program_id(1)
    nb = pl.num_programs(1) - 1
    bm = x_ref.shape[1]

    @pl.when(tb == 0)
    def _init():
        h_ref[...] = jnp.zeros_like(h_ref)
        x0 = x_ref[0]
        ema1_ref[...] = x0
        ema2_ref[...] = x0

    sel_w = lax.rem(tb, 2)
    sel_r = lax.rem(tb + 1, 2)

    e1_in = ema1_ref[...]
    e2_in = ema2_ref[...]

    def body(t, carry):
        prev1, prev2, h = carry

        xt = x_ref[t]
        cur1 = (1.0 - _ALPHA) * xt + _ALPHA * prev1
        cur2 = (1.0 - _BETA) * cur1 + _BETA * prev2
        xs2_ref[t] = cur2

        gi = xp_ref[sel_r, pl.ds(t * bm, bm), :]
        gh = jnp.dot(h, whh_ref[...], preferred_element_type=jnp.float32)
        r = _sigmoid(gi[:, 0:128] + gh[:, 0:128])
        z = _sigmoid(gi[:, 128:256] + gh[:, 128:256])
        n = jnp.tanh(gi[:, 256:384] + r * (gh[:, 256:384] + bhhn_ref[...]))
        hn = n + z * (h - n)

        return (cur1, cur2, hn)

    p1, p2, h = lax.fori_loop(0, _TB, body, (e1_in, e2_in, h_ref[...]),
                              unroll=8)

    @pl.when(tb < nb)
    def _commit_ema():
        ema1_ref[...] = p1
        ema2_ref[...] = p2

    h_ref[...] = h

    @pl.when(tb == 0)
    def _rezero_h():
        h_ref[...] = jnp.zeros_like(h_ref)

    @pl.when(tb < nb)
    def _project():
        xs2 = xs2_ref[...].reshape(_TB * bm, x_ref.shape[2])
        xp_ref[sel_w] = (
            jnp.dot(xs2, wih_ref[...], preferred_element_type=jnp.float32)
            + bproj_ref[...]
        )

    @pl.when(tb == nb)
    def _final():
        out = jnp.dot(h, wfc_ref[...], preferred_element_type=jnp.float32) \
            + bfc_ref[...]
        st11 = e1_in[:, 1:3]
        st12 = e2_in[:, 1:3]
        out = (out - _BETA * st12) / (1.0 - _BETA)
        out = (out - _ALPHA * st11) / (1.0 - _ALPHA)
        out_ref[...] = out


def kernel(x, W_ih, W_hh, b_ih, b_hh, W_fc, b_fc):
    B, T, I = x.shape
    H = W_hh.shape[1]
    C = W_fc.shape[0]
    nb = T // _TB
    xt = jnp.swapaxes(x, 0, 1)

    b_proj = (b_ih + jnp.concatenate([b_hh[:2 * H], jnp.zeros((H,), b_hh.dtype)]))
    b_hh_n = b_hh[2 * H:]

    out = pl.pallas_call(
        _fused_kernel,
        grid=(2, nb + 1),
        in_specs=[
            pl.BlockSpec((_TB, B // 2, I),
                         lambda c, t: (jnp.minimum(t, nb - 1), c, 0)),
            pl.BlockSpec((I, 3 * H), lambda c, t: (0, 0)),
            pl.BlockSpec((H, 3 * H), lambda c, t: (0, 0)),
            pl.BlockSpec((1, 3 * H), lambda c, t: (0, 0)),
            pl.BlockSpec((1, H), lambda c, t: (0, 0)),
            pl.BlockSpec((H, C), lambda c, t: (0, 0)),
            pl.BlockSpec((1, C), lambda c, t: (0, 0)),
        ],
        out_specs=pl.BlockSpec((B // 2, C), lambda c, t: (c, 0)),
        out_shape=jax.ShapeDtypeStruct((B, C), jnp.float32),
        scratch_shapes=[
            pltpu.VMEM((B // 2, I), jnp.float32),
            pltpu.VMEM((B // 2, I), jnp.float32),
            pltpu.VMEM((B // 2, H), jnp.float32),
            pltpu.VMEM((_TB, B // 2, I), jnp.float32),
            pltpu.VMEM((2, _TB * (B // 2), 3 * H), jnp.float32),
        ],
        compiler_params=pltpu.CompilerParams(
            dimension_semantics=("parallel", "arbitrary"),
            vmem_limit_bytes=100 * 1024 * 1024,
        ),
    )(xt, W_ih.T, W_hh.T, b_proj.reshape(1, -1), b_hh_n.reshape(1, -1),
      W_fc.T, b_fc.reshape(1, -1))

    return out[:, None, :]
